# baseline (speedup 1.0000x reference)
import jax
import jax.numpy as jnp
from jax import lax
from jax.experimental import pallas as pl
from jax.experimental.pallas import tpu as pltpu

_B, _N, _C = 4, 5000, 21
_NP = 5120
_NG = _NP // 16
_BLK = 128
_RED = 16.0
_IOU_T = 0.5


def _body(nms_reg_ref, rcnn_reg_ref, rcnn_cls_ref,
          reg_adj_ref, cls_sm_ref, keep_ref,
          t_ref, rows_ref, vcol_ref, crow_ref, vrow_ref):
    nr = nms_reg_ref[0]
    rounded = jnp.concatenate(
        [jnp.floor(nr[0:2, :] * _RED), jnp.ceil(nr[2:4, :] * _RED)],
        axis=0) / _RED
    adj = rcnn_reg_ref[0] + rounded
    reg_adj_ref[0] = adj

    logits = rcnn_cls_ref[0]
    lmax = jnp.max(logits, axis=0, keepdims=True)
    ex = jnp.exp(logits - lmax)
    sm = ex / jnp.sum(ex, axis=0, keepdims=True)
    cls_sm_ref[0] = sm

    maxp = jnp.max(sm, axis=0, keepdims=True)
    cidx = lax.broadcasted_iota(jnp.int32, (_C, _NP), 0)
    amax = jnp.min(jnp.where(sm == maxp, cidx, jnp.int32(_C)),
                   axis=0, keepdims=True).astype(jnp.float32)
    fgr = (amax != 0.0).astype(jnp.float32)
    bidx = lax.broadcasted_iota(jnp.int32, (1, _NP), 1)
    key = jnp.where(amax == 0.0, -(bidx.astype(jnp.float32) + 1.0), amax)
    arear = (jnp.maximum(adj[2:3, :] - adj[0:1, :], 0.0)
             * jnp.maximum(adj[3:4, :] - adj[1:2, :], 0.0))

    rows_ref[0:1, :] = maxp
    rows_ref[1:2, :] = key
    rows_ref[2:3, :] = arear
    rows_ref[3:4, :] = fgr

    wl = lax.broadcasted_iota(jnp.int32, (8, _BLK), 1)
    wm = lax.broadcasted_iota(jnp.int32, (8, _BLK), 0)
    w_pack = jnp.where((wl >> 4) == wm,
                       (1 << (wl & 15)).astype(jnp.float32), 0.0)

    def j_body(jb, _):
        j0 = jb * _BLK

        def col(row_chunk):
            return row_chunk.reshape(_BLK, 1)

        y1j = col(reg_adj_ref[0, 0:1, pl.ds(j0, _BLK)])
        x1j = col(reg_adj_ref[0, 1:2, pl.ds(j0, _BLK)])
        y2j = col(reg_adj_ref[0, 2:3, pl.ds(j0, _BLK)])
        x2j = col(reg_adj_ref[0, 3:4, pl.ds(j0, _BLK)])
        scj = col(rows_ref[0:1, pl.ds(j0, _BLK)])
        kyj = col(rows_ref[1:2, pl.ds(j0, _BLK)])
        arj = col(rows_ref[2:3, pl.ds(j0, _BLK)])
        jidx = j0 + lax.broadcasted_iota(jnp.int32, (_BLK, 1), 0)

        def i_body(ic, _):
            i0 = ic * _BLK
            y1i = reg_adj_ref[0, 0:1, pl.ds(i0, _BLK)]
            x1i = reg_adj_ref[0, 1:2, pl.ds(i0, _BLK)]
            y2i = reg_adj_ref[0, 2:3, pl.ds(i0, _BLK)]
            x2i = reg_adj_ref[0, 3:4, pl.ds(i0, _BLK)]
            sci = rows_ref[0:1, pl.ds(i0, _BLK)]
            kyi = rows_ref[1:2, pl.ds(i0, _BLK)]
            ari = rows_ref[2:3, pl.ds(i0, _BLK)]
            iidx = i0 + lax.broadcasted_iota(jnp.int32, (1, _BLK), 1)

            ih = jnp.maximum(jnp.minimum(y2j, y2i) - jnp.maximum(y1j, y1i), 0.0)
            iw = jnp.maximum(jnp.minimum(x2j, x2i) - jnp.maximum(x1j, x1i), 0.0)
            inter = ih * iw
            union = arj + ari - inter
            ov = inter > _IOU_T * jnp.maximum(union, 1e-9)
            prec = (scj > sci) | ((scj == sci) & (jidx < iidx))
            okf = jnp.where(ov & prec & (kyj == kyi), 1.0, 0.0)
            packed = jnp.dot(w_pack, okf,
                             preferred_element_type=jnp.float32)
            t_ref[pl.ds(jb * 8, 8), pl.ds(i0, _BLK)] = packed.astype(jnp.int32)
            return 0

        lax.fori_loop(0, _NP // _BLK, i_body, 0)
        return 0

    lax.fori_loop(0, _NP // _BLK, j_body, 0)

    def matvec(vec_row):
        vrow_ref[...] = vec_row

        def pack_b(jb, _):
            chunk = vrow_ref[0:1, pl.ds(jb * _BLK, _BLK)]
            vp8 = jnp.dot(w_pack, chunk.reshape(_BLK, 1),
                          preferred_element_type=jnp.float32)
            vcol_ref[pl.ds(jb * 8, 8), 0:1] = vp8.astype(jnp.int32)
            return 0

        lax.fori_loop(0, _NP // _BLK, pack_b, 0)

        def mv_c(c, _):
            i0 = c * 512

            def mv_g(gb, racc):
                blk = t_ref[pl.ds(gb * 8, 8), pl.ds(i0, 512)]
                vp = vcol_ref[pl.ds(gb * 8, 8), 0:1]
                return racc | (blk & vp)

            racc = lax.fori_loop(0, _NG // 8, mv_g,
                                 jnp.zeros((8, 512), jnp.int32))
            hit = jnp.max(racc, axis=0, keepdims=True)
            crow_ref[0:1, pl.ds(i0, 512)] = (hit != 0).astype(jnp.float32)
            return 0

        lax.fori_loop(0, _NP // 512, mv_c, 0)
        return crow_ref[...]

    active0 = rows_ref[3:4, :]
    kept0 = jnp.zeros((1, _NP), jnp.float32)

    def cond(state):
        r, active, _ = state
        return (jnp.max(active) > 0.0) & (r < _NP)

    def round_body(state):
        r, active, kept = state
        conflict = matvec(active)
        can_keep = active * (1.0 - conflict)
        sup = matvec(can_keep)
        kept = jnp.maximum(kept, can_keep)
        active = active * (1.0 - can_keep) * (1.0 - sup)
        return r + 1, active, kept

    _, _, kept = lax.while_loop(cond, round_body, (0, active0, kept0))
    keep_ref[0] = kept


def _pipeline(nms_reg, rcnn_reg, rcnn_cls):
    padn = _NP - _N
    nr = jnp.transpose(jnp.pad(nms_reg, ((0, 0), (0, padn), (0, 0))), (0, 2, 1))
    rr = jnp.transpose(jnp.pad(rcnn_reg, ((0, 0), (0, padn), (0, 0))), (0, 2, 1))
    rc = jnp.transpose(jnp.pad(rcnn_cls, ((0, 0), (0, padn), (0, 0))), (0, 2, 1))

    out = pl.pallas_call(
        _body,
        grid=(_B,),
        in_specs=[
            pl.BlockSpec((1, 4, _NP), lambda b: (b, 0, 0)),
            pl.BlockSpec((1, 4, _NP), lambda b: (b, 0, 0)),
            pl.BlockSpec((1, _C, _NP), lambda b: (b, 0, 0)),
        ],
        out_specs=[
            pl.BlockSpec((1, 4, _NP), lambda b: (b, 0, 0)),
            pl.BlockSpec((1, _C, _NP), lambda b: (b, 0, 0)),
            pl.BlockSpec((1, 1, _NP), lambda b: (b, 0, 0)),
        ],
        out_shape=[
            jax.ShapeDtypeStruct((_B, 4, _NP), jnp.float32),
            jax.ShapeDtypeStruct((_B, _C, _NP), jnp.float32),
            jax.ShapeDtypeStruct((_B, 1, _NP), jnp.float32),
        ],
        scratch_shapes=[
            pltpu.VMEM((_NG, _NP), jnp.int32),
            pltpu.VMEM((8, _NP), jnp.float32),
            pltpu.VMEM((_NG, 1), jnp.int32),
            pltpu.VMEM((1, _NP), jnp.float32),
            pltpu.VMEM((1, _NP), jnp.float32),
        ],
    )(nr, rr, rc)
    return out


def kernel(nms_reg, nms_cls, rcnn_reg, rcnn_cls):
    reg_adj, cls_sm, keep = _pipeline(nms_reg, rcnn_reg, rcnn_cls)
    return (nms_reg, nms_cls,
            reg_adj,
            cls_sm,
            keep[:, 0, :_N] > 0.5)


# speedup vs baseline: 1.7384x; 1.7384x over previous
import jax
import jax.numpy as jnp
from jax import lax
from jax.experimental import pallas as pl
from jax.experimental.pallas import tpu as pltpu

_B, _N, _C = 4, 5000, 21
_NP = 5120
_NG = _NP // 16
_BLK = 128
_RED = 16.0
_IOU_T = 0.5


def _body(nms_reg_ref, rcnn_reg_ref, rcnn_cls_ref,
          reg_adj_ref, cls_sm_ref, keep_ref,
          t_ref, rows_ref, vcol_ref, crow_ref, vrow_ref):
    nr = nms_reg_ref[0]
    rounded = jnp.concatenate(
        [jnp.floor(nr[0:2, :] * _RED), jnp.ceil(nr[2:4, :] * _RED)],
        axis=0) / _RED
    adj = rcnn_reg_ref[0] + rounded
    reg_adj_ref[0] = adj

    logits = rcnn_cls_ref[0]
    lmax = jnp.max(logits, axis=0, keepdims=True)
    ex = jnp.exp(logits - lmax)
    sm = ex / jnp.sum(ex, axis=0, keepdims=True)
    cls_sm_ref[0] = sm

    maxp = jnp.max(sm, axis=0, keepdims=True)
    cidx = lax.broadcasted_iota(jnp.int32, (_C, _NP), 0)
    amax = jnp.min(jnp.where(sm == maxp, cidx, jnp.int32(_C)),
                   axis=0, keepdims=True).astype(jnp.float32)
    fgr = (amax != 0.0).astype(jnp.float32)
    bidx = lax.broadcasted_iota(jnp.int32, (1, _NP), 1)
    key = jnp.where(amax == 0.0, -(bidx.astype(jnp.float32) + 1.0), amax)
    arear = (jnp.maximum(adj[2:3, :] - adj[0:1, :], 0.0)
             * jnp.maximum(adj[3:4, :] - adj[1:2, :], 0.0))

    rows_ref[0:1, :] = maxp
    rows_ref[1:2, :] = key
    rows_ref[2:3, :] = arear
    rows_ref[3:4, :] = fgr

    wl = lax.broadcasted_iota(jnp.int32, (8, _BLK), 1)
    wm = lax.broadcasted_iota(jnp.int32, (8, _BLK), 0)
    w_pack = jnp.where((wl >> 4) == wm,
                       (1 << (wl & 15)).astype(jnp.float32), 0.0)
    w_packb = w_pack.astype(jnp.bfloat16)

    def j_body(jb, _):
        j0 = jb * _BLK

        def col(row_chunk):
            return row_chunk.reshape(_BLK, 1)

        y1j = col(reg_adj_ref[0, 0:1, pl.ds(j0, _BLK)])
        x1j = col(reg_adj_ref[0, 1:2, pl.ds(j0, _BLK)])
        y2j = col(reg_adj_ref[0, 2:3, pl.ds(j0, _BLK)])
        x2j = col(reg_adj_ref[0, 3:4, pl.ds(j0, _BLK)])
        scj = col(rows_ref[0:1, pl.ds(j0, _BLK)])
        kyj = col(rows_ref[1:2, pl.ds(j0, _BLK)])
        arj = col(rows_ref[2:3, pl.ds(j0, _BLK)])
        jidx = j0 + lax.broadcasted_iota(jnp.int32, (_BLK, 1), 0)

        def one_iblock(i0):
            y1i = reg_adj_ref[0, 0:1, pl.ds(i0, _BLK)]
            x1i = reg_adj_ref[0, 1:2, pl.ds(i0, _BLK)]
            y2i = reg_adj_ref[0, 2:3, pl.ds(i0, _BLK)]
            x2i = reg_adj_ref[0, 3:4, pl.ds(i0, _BLK)]
            sci = rows_ref[0:1, pl.ds(i0, _BLK)]
            kyi = rows_ref[1:2, pl.ds(i0, _BLK)]
            ari = rows_ref[2:3, pl.ds(i0, _BLK)]
            iidx = i0 + lax.broadcasted_iota(jnp.int32, (1, _BLK), 1)

            ih = jnp.maximum(jnp.minimum(y2j, y2i) - jnp.maximum(y1j, y1i), 0.0)
            iw = jnp.maximum(jnp.minimum(x2j, x2i) - jnp.maximum(x1j, x1i), 0.0)
            inter = ih * iw
            union = arj + ari - inter
            ov = inter > _IOU_T * jnp.maximum(union, 1e-9)
            prec = (scj > sci) | ((scj == sci) & (jidx < iidx))
            okb = jnp.where(ov & prec & (kyj == kyi),
                            1.0, 0.0).astype(jnp.bfloat16)
            packed = jnp.dot(w_packb, okb,
                             preferred_element_type=jnp.float32)
            t_ref[pl.ds(jb * 8, 8), pl.ds(i0, _BLK)] = packed.astype(jnp.int32)

        def i_body(ic, _):
            one_iblock(ic * (2 * _BLK))
            one_iblock(ic * (2 * _BLK) + _BLK)
            return 0

        lax.fori_loop(0, _NP // (2 * _BLK), i_body, 0)
        return 0

    lax.fori_loop(0, _NP // _BLK, j_body, 0)

    def matvec(vec_row):
        vrow_ref[...] = vec_row

        def pack_b(jb, _):
            chunk = vrow_ref[0:1, pl.ds(jb * _BLK, _BLK)]
            vp8 = jnp.dot(w_packb, chunk.reshape(_BLK, 1).astype(jnp.bfloat16),
                          preferred_element_type=jnp.float32)
            vcol_ref[pl.ds(jb * 8, 8), 0:1] = vp8.astype(jnp.int32)
            return 0

        lax.fori_loop(0, _NP // _BLK, pack_b, 0)

        def mv_c(c, _):
            i0 = c * 512

            def mv_g(gb, racc):
                blk = t_ref[pl.ds(gb * 32, 32), pl.ds(i0, 512)]
                vp = vcol_ref[pl.ds(gb * 32, 32), 0:1]
                return jnp.maximum(racc,
                                   jnp.max(blk & vp, axis=0, keepdims=True))

            racc = lax.fori_loop(0, _NG // 32, mv_g,
                                 jnp.zeros((1, 512), jnp.int32))
            crow_ref[0:1, pl.ds(i0, 512)] = (racc != 0).astype(jnp.float32)
            return 0

        lax.fori_loop(0, _NP // 512, mv_c, 0)
        return crow_ref[...]

    active0 = rows_ref[3:4, :]
    kept0 = jnp.zeros((1, _NP), jnp.float32)

    def cond(state):
        r, active, _ = state
        return (jnp.max(active) > 0.0) & (r < _NP)

    def round_body(state):
        r, active, kept = state
        conflict = matvec(active)
        can_keep = active * (1.0 - conflict)
        sup = matvec(can_keep)
        kept = jnp.maximum(kept, can_keep)
        active = active * (1.0 - can_keep) * (1.0 - sup)
        return r + 1, active, kept

    _, _, kept = lax.while_loop(cond, round_body, (0, active0, kept0))
    keep_ref[0] = kept


def _pipeline(nms_reg, rcnn_reg, rcnn_cls):
    padn = _NP - _N
    nr = jnp.transpose(jnp.pad(nms_reg, ((0, 0), (0, padn), (0, 0))), (0, 2, 1))
    rr = jnp.transpose(jnp.pad(rcnn_reg, ((0, 0), (0, padn), (0, 0))), (0, 2, 1))
    rc = jnp.transpose(jnp.pad(rcnn_cls, ((0, 0), (0, padn), (0, 0))), (0, 2, 1))

    out = pl.pallas_call(
        _body,
        grid=(_B,),
        in_specs=[
            pl.BlockSpec((1, 4, _NP), lambda b: (b, 0, 0)),
            pl.BlockSpec((1, 4, _NP), lambda b: (b, 0, 0)),
            pl.BlockSpec((1, _C, _NP), lambda b: (b, 0, 0)),
        ],
        out_specs=[
            pl.BlockSpec((1, 4, _NP), lambda b: (b, 0, 0)),
            pl.BlockSpec((1, _C, _NP), lambda b: (b, 0, 0)),
            pl.BlockSpec((1, 1, _NP), lambda b: (b, 0, 0)),
        ],
        out_shape=[
            jax.ShapeDtypeStruct((_B, 4, _NP), jnp.float32),
            jax.ShapeDtypeStruct((_B, _C, _NP), jnp.float32),
            jax.ShapeDtypeStruct((_B, 1, _NP), jnp.float32),
        ],
        scratch_shapes=[
            pltpu.VMEM((_NG, _NP), jnp.int32),
            pltpu.VMEM((8, _NP), jnp.float32),
            pltpu.VMEM((_NG, 1), jnp.int32),
            pltpu.VMEM((1, _NP), jnp.float32),
            pltpu.VMEM((1, _NP), jnp.float32),
        ],
    )(nr, rr, rc)
    return out


def kernel(nms_reg, nms_cls, rcnn_reg, rcnn_cls):
    reg_adj, cls_sm, keep = _pipeline(nms_reg, rcnn_reg, rcnn_cls)
    return (nms_reg, nms_cls,
            jnp.transpose(reg_adj, (0, 2, 1))[:, :_N, :],
            jnp.transpose(cls_sm, (0, 2, 1))[:, :_N, :],
            keep[:, 0, :_N] > 0.5)
